# initial kernel scaffold (unmeasured)
import jax
import jax.numpy as jnp
from jax import lax
from jax.experimental import pallas as pl
from jax.experimental.pallas import tpu as pltpu

N_DEV = 4


def kernel(x, w_mat):
    m_per, k = x.shape
    k2, n_per = w_mat.shape

    def body(x_ref, w_ref, out_ref, comm_ref, send_sems, recv_sems):
        my_pos = lax.axis_index("i")
        left = (my_pos - 1) % N_DEV
        right = (my_pos + 1) % N_DEV

        barrier_sem = pltpu.get_barrier_semaphore()
        for nbr in [left, right]:
            pl.semaphore_signal(
                barrier_sem, inc=1,
                device_id=(nbr,), device_id_type=pl.DeviceIdType.MESH,
            )
        pl.semaphore_wait(barrier_sem, 2)

        out_ref[pl.ds(my_pos * m_per, m_per), :] = jnp.dot(
            x_ref[:, :], w_ref[:, :], preferred_element_type=jnp.float32
        )

        for h in range(N_DEV - 1):
            src = x_ref if h == 0 else comm_ref.at[h - 1]
            rdma = pltpu.make_async_remote_copy(
                src_ref=src,
                dst_ref=comm_ref.at[h],
                send_sem=send_sems.at[h],
                recv_sem=recv_sems.at[h],
                device_id=(right,),
                device_id_type=pl.DeviceIdType.MESH,
            )
            rdma.start()
            rdma.wait()

            origin = (my_pos - h - 1) % N_DEV
            out_ref[pl.ds(origin * m_per, m_per), :] = jnp.dot(
                comm_ref[h, :, :], w_ref[:, :],
                preferred_element_type=jnp.float32,
            )

    return pl.pallas_call(
        body,
        out_shape=jax.ShapeDtypeStruct((N_DEV * m_per, n_per), jnp.float32),
        in_specs=[
            pl.BlockSpec(memory_space=pltpu.VMEM),
            pl.BlockSpec(memory_space=pltpu.VMEM),
        ],
        out_specs=pl.BlockSpec(memory_space=pltpu.VMEM),
        scratch_shapes=[
            pltpu.VMEM((N_DEV - 1, m_per, k), jnp.float32),
            pltpu.SemaphoreType.DMA((N_DEV - 1,)),
            pltpu.SemaphoreType.DMA((N_DEV - 1,)),
        ],
        compiler_params=pltpu.CompilerParams(collective_id=0),
    )(x, w_mat)


# baseline (device time: 580009 ns/iter reference)
import jax
import jax.numpy as jnp
from jax import lax
from jax.experimental import pallas as pl
from jax.experimental.pallas import tpu as pltpu

N_DEV = 4


def kernel(x, w_mat):
    m_per, k = x.shape
    k2, n_per = w_mat.shape

    def body(x_hbm, w_ref, out_ref, comm_ref, send_sems, recv_sems, copy_sem):
        my_pos = lax.axis_index("i")
        left = (my_pos - 1) % N_DEV
        right = (my_pos + 1) % N_DEV

        barrier_sem = pltpu.get_barrier_semaphore()
        for nbr in [left, right]:
            pl.semaphore_signal(
                barrier_sem, inc=1,
                device_id=(nbr,), device_id_type=pl.DeviceIdType.MESH,
            )
        pl.semaphore_wait(barrier_sem, 2)

        cp = pltpu.make_async_copy(x_hbm, comm_ref.at[0], copy_sem)
        cp.start()
        cp.wait()

        out_ref[pl.ds(my_pos * m_per, m_per), :] = jnp.dot(
            comm_ref[0, :, :], w_ref[:, :], preferred_element_type=jnp.float32
        )

        for h in range(N_DEV - 1):
            send_slot = h % 2
            recv_slot = (h + 1) % 2
            rdma = pltpu.make_async_remote_copy(
                src_ref=comm_ref.at[send_slot],
                dst_ref=comm_ref.at[recv_slot],
                send_sem=send_sems.at[h],
                recv_sem=recv_sems.at[h],
                device_id=(right,),
                device_id_type=pl.DeviceIdType.MESH,
            )
            rdma.start()
            rdma.wait()

            origin = (my_pos - h - 1) % N_DEV
            out_ref[pl.ds(origin * m_per, m_per), :] = jnp.dot(
                comm_ref[recv_slot, :, :], w_ref[:, :],
                preferred_element_type=jnp.float32,
            )

    return pl.pallas_call(
        body,
        out_shape=jax.ShapeDtypeStruct((N_DEV * m_per, n_per), jnp.float32),
        in_specs=[
            pl.BlockSpec(memory_space=pl.ANY),
            pl.BlockSpec(memory_space=pltpu.VMEM),
        ],
        out_specs=pl.BlockSpec(memory_space=pltpu.VMEM),
        scratch_shapes=[
            pltpu.VMEM((2, m_per, k), jnp.float32),
            pltpu.SemaphoreType.DMA((N_DEV - 1,)),
            pltpu.SemaphoreType.DMA((N_DEV - 1,)),
            pltpu.SemaphoreType.DMA,
        ],
        compiler_params=pltpu.CompilerParams(collective_id=0),
    )(x, w_mat)


# device time: 296945 ns/iter; 1.9533x vs baseline; 1.9533x over previous
import jax
import jax.numpy as jnp
from jax import lax
from jax.experimental import pallas as pl
from jax.experimental.pallas import tpu as pltpu

N_DEV = 4


def kernel(x, w_mat):
    m_per, k = x.shape
    k2, n_per = w_mat.shape
    half = m_per // 2

    def body(x_hbm, w_ref, out_ref, comm_r, comm_l,
             send_r, recv_r, send_l, recv_l,
             credit_r, credit_l, copy_sems):
        my = lax.axis_index("i")
        left = (my - 1) % N_DEV
        right = (my + 1) % N_DEV

        barrier_sem = pltpu.get_barrier_semaphore()
        for nbr in [left, right]:
            pl.semaphore_signal(
                barrier_sem, inc=1,
                device_id=(nbr,), device_id_type=pl.DeviceIdType.MESH,
            )
        pl.semaphore_wait(barrier_sem, 2)

        cp0 = pltpu.make_async_copy(
            x_hbm.at[pl.ds(0, half)], comm_r.at[0], copy_sems.at[0])
        cp1 = pltpu.make_async_copy(
            x_hbm.at[pl.ds(half, half)], comm_l.at[0], copy_sems.at[1])
        cp0.start()
        cp1.start()
        cp0.wait()
        cp1.wait()

        def mk(comm, sends, recvs, sslot, rslot, h, dst):
            return pltpu.make_async_remote_copy(
                src_ref=comm.at[sslot],
                dst_ref=comm.at[rslot],
                send_sem=sends.at[h],
                recv_sem=recvs.at[h],
                device_id=(dst,),
                device_id_type=pl.DeviceIdType.MESH,
            )

        def mk_r(sslot, rslot, h):
            return mk(comm_r, send_r, recv_r, sslot, rslot, h, right)

        def mk_l(sslot, rslot, h):
            return mk(comm_l, send_l, recv_l, sslot, rslot, h, left)

        mk_r(0, 1, 0).start()
        mk_l(0, 1, 0).start()

        out_ref[pl.ds(my * m_per, half), :] = jnp.dot(
            comm_r[0], w_ref[:, :], preferred_element_type=jnp.float32)
        out_ref[pl.ds(my * m_per + half, half), :] = jnp.dot(
            comm_l[0], w_ref[:, :], preferred_element_type=jnp.float32)

        for h in range(N_DEV - 1):
            sslot, rslot = h % 2, (h + 1) % 2
            r = mk_r(sslot, rslot, h)
            l = mk_l(sslot, rslot, h)
            r.wait_recv()
            l.wait_recv()
            r.wait_send()
            l.wait_send()
            if h < N_DEV - 2:
                pl.semaphore_signal(credit_r, inc=1, device_id=(left,),
                                    device_id_type=pl.DeviceIdType.MESH)
                pl.semaphore_signal(credit_l, inc=1, device_id=(right,),
                                    device_id_type=pl.DeviceIdType.MESH)
                pl.semaphore_wait(credit_r, 1)
                pl.semaphore_wait(credit_l, 1)
                mk_r(rslot, sslot, h + 1).start()
                mk_l(rslot, sslot, h + 1).start()
            o_r = (my - h - 1) % N_DEV
            o_l = (my + h + 1) % N_DEV
            out_ref[pl.ds(o_r * m_per, half), :] = jnp.dot(
                comm_r[rslot], w_ref[:, :],
                preferred_element_type=jnp.float32)
            out_ref[pl.ds(o_l * m_per + half, half), :] = jnp.dot(
                comm_l[rslot], w_ref[:, :],
                preferred_element_type=jnp.float32)

    return pl.pallas_call(
        body,
        out_shape=jax.ShapeDtypeStruct((N_DEV * m_per, n_per), jnp.float32),
        in_specs=[
            pl.BlockSpec(memory_space=pl.ANY),
            pl.BlockSpec(memory_space=pltpu.VMEM),
        ],
        out_specs=pl.BlockSpec(memory_space=pltpu.VMEM),
        scratch_shapes=[
            pltpu.VMEM((2, half, k), jnp.float32),
            pltpu.VMEM((2, half, k), jnp.float32),
            pltpu.SemaphoreType.DMA((N_DEV - 1,)),
            pltpu.SemaphoreType.DMA((N_DEV - 1,)),
            pltpu.SemaphoreType.DMA((N_DEV - 1,)),
            pltpu.SemaphoreType.DMA((N_DEV - 1,)),
            pltpu.SemaphoreType.REGULAR,
            pltpu.SemaphoreType.REGULAR,
            pltpu.SemaphoreType.DMA((2,)),
        ],
        compiler_params=pltpu.CompilerParams(collective_id=0),
    )(x, w_mat)


# device time: 163259 ns/iter; 3.5527x vs baseline; 1.8189x over previous
import jax
import jax.numpy as jnp
from jax import lax
from jax.experimental import pallas as pl
from jax.experimental.pallas import tpu as pltpu

N_DEV = 4


def kernel(x, w_mat):
    m_per, k = x.shape
    k2, n_per = w_mat.shape
    half = m_per // 2

    xb = x.astype(jnp.bfloat16)

    def body(x_ref, w_ref, out_ref, comm_r, comm_l,
             send_r, recv_r, send_l, recv_l,
             credit_r, credit_l):
        my = lax.axis_index("i")
        left = (my - 1) % N_DEV
        right = (my + 1) % N_DEV

        barrier_sem = pltpu.get_barrier_semaphore()
        for nbr in [left, right]:
            pl.semaphore_signal(
                barrier_sem, inc=1,
                device_id=(nbr,), device_id_type=pl.DeviceIdType.MESH,
            )
        pl.semaphore_wait(barrier_sem, 2)

        comm_r[0] = x_ref[pl.ds(0, half), :]
        comm_l[0] = x_ref[pl.ds(half, half), :]

        def mk(comm, sends, recvs, sslot, rslot, h, dst):
            return pltpu.make_async_remote_copy(
                src_ref=comm.at[sslot],
                dst_ref=comm.at[rslot],
                send_sem=sends.at[h],
                recv_sem=recvs.at[h],
                device_id=(dst,),
                device_id_type=pl.DeviceIdType.MESH,
            )

        def mk_r(sslot, rslot, h):
            return mk(comm_r, send_r, recv_r, sslot, rslot, h, right)

        def mk_l(sslot, rslot, h):
            return mk(comm_l, send_l, recv_l, sslot, rslot, h, left)

        mk_r(0, 1, 0).start()
        mk_l(0, 1, 0).start()

        out_ref[pl.ds(my * m_per, half), :] = jnp.dot(
            comm_r[0], w_ref[:, :], preferred_element_type=jnp.float32)
        out_ref[pl.ds(my * m_per + half, half), :] = jnp.dot(
            comm_l[0], w_ref[:, :], preferred_element_type=jnp.float32)

        for h in range(N_DEV - 1):
            sslot, rslot = h % 2, (h + 1) % 2
            r = mk_r(sslot, rslot, h)
            l = mk_l(sslot, rslot, h)
            r.wait_recv()
            l.wait_recv()
            r.wait_send()
            l.wait_send()
            if h < N_DEV - 2:
                pl.semaphore_signal(credit_r, inc=1, device_id=(left,),
                                    device_id_type=pl.DeviceIdType.MESH)
                pl.semaphore_signal(credit_l, inc=1, device_id=(right,),
                                    device_id_type=pl.DeviceIdType.MESH)
                pl.semaphore_wait(credit_r, 1)
                pl.semaphore_wait(credit_l, 1)
                mk_r(rslot, sslot, h + 1).start()
                mk_l(rslot, sslot, h + 1).start()
            o_r = (my - h - 1) % N_DEV
            o_l = (my + h + 1) % N_DEV
            out_ref[pl.ds(o_r * m_per, half), :] = jnp.dot(
                comm_r[rslot], w_ref[:, :],
                preferred_element_type=jnp.float32)
            out_ref[pl.ds(o_l * m_per + half, half), :] = jnp.dot(
                comm_l[rslot], w_ref[:, :],
                preferred_element_type=jnp.float32)

    return pl.pallas_call(
        body,
        out_shape=jax.ShapeDtypeStruct((N_DEV * m_per, n_per), jnp.float32),
        in_specs=[
            pl.BlockSpec(memory_space=pltpu.VMEM),
            pl.BlockSpec(memory_space=pltpu.VMEM),
        ],
        out_specs=pl.BlockSpec(memory_space=pltpu.VMEM),
        scratch_shapes=[
            pltpu.VMEM((2, half, k), jnp.bfloat16),
            pltpu.VMEM((2, half, k), jnp.bfloat16),
            pltpu.SemaphoreType.DMA((N_DEV - 1,)),
            pltpu.SemaphoreType.DMA((N_DEV - 1,)),
            pltpu.SemaphoreType.DMA((N_DEV - 1,)),
            pltpu.SemaphoreType.DMA((N_DEV - 1,)),
            pltpu.SemaphoreType.REGULAR,
            pltpu.SemaphoreType.REGULAR,
        ],
        compiler_params=pltpu.CompilerParams(collective_id=0),
    )(xb, w_mat)


# device time: 134488 ns/iter; 4.3127x vs baseline; 1.2139x over previous
import jax
import jax.numpy as jnp
from jax import lax
from jax.experimental import pallas as pl
from jax.experimental.pallas import tpu as pltpu

N_DEV = 4


def kernel(x, w_mat):
    m_per, k = x.shape
    k2, n_per = w_mat.shape
    nh = n_per // 2

    xb = x.astype(jnp.bfloat16)
    wb = w_mat.astype(jnp.bfloat16)

    def body(x_ref, w_ref, out_ref, comm_r, comm_l, tile_bufs,
             send_r, recv_r, send_l, recv_l,
             credit_r, credit_l, tile_send, tile_recv):
        my = lax.axis_index("i")
        left = (my - 1) % N_DEV
        right = (my + 1) % N_DEV
        diag = (my + 2) % N_DEV

        barrier_sem = pltpu.get_barrier_semaphore()
        for nbr in [left, right, diag]:
            pl.semaphore_signal(
                barrier_sem, inc=1,
                device_id=(nbr,), device_id_type=pl.DeviceIdType.MESH,
            )
        pl.semaphore_wait(barrier_sem, 3)

        comm_r[0] = w_ref[:, pl.ds(0, nh)]
        comm_l[0] = w_ref[:, pl.ds(nh, nh)]

        def mk(comm, sends, recvs, sslot, rslot, h, dst):
            return pltpu.make_async_remote_copy(
                src_ref=comm.at[sslot],
                dst_ref=comm.at[rslot],
                send_sem=sends.at[h],
                recv_sem=recvs.at[h],
                device_id=(dst,),
                device_id_type=pl.DeviceIdType.MESH,
            )

        def mk_r(sslot, rslot, h):
            return mk(comm_r, send_r, recv_r, sslot, rslot, h, right)

        def mk_l(sslot, rslot, h):
            return mk(comm_l, send_l, recv_l, sslot, rslot, h, left)

        def send_tile(src, dst_rows, dst_col, dst_w, slot, dst_dev):
            t = pltpu.make_async_remote_copy(
                src_ref=src,
                dst_ref=out_ref.at[pl.ds(dst_rows * m_per, m_per),
                                   pl.ds(dst_col, dst_w)],
                send_sem=tile_send.at[slot],
                recv_sem=tile_recv.at[slot],
                device_id=(dst_dev,),
                device_id_type=pl.DeviceIdType.MESH,
            )
            t.start()
            return t

        mk_r(0, 1, 0).start()
        mk_l(0, 1, 0).start()

        out_ref[pl.ds(my * m_per, m_per), :] = jnp.dot(
            x_ref[:, :], w_ref[:, :], preferred_element_type=jnp.float32)

        for h in range(N_DEV - 1):
            sslot, rslot = h % 2, (h + 1) % 2
            r = mk_r(sslot, rslot, h)
            l = mk_l(sslot, rslot, h)
            r.wait_recv()
            l.wait_recv()
            r.wait_send()
            l.wait_send()
            if h < N_DEV - 2:
                pl.semaphore_signal(credit_r, inc=1, device_id=(left,),
                                    device_id_type=pl.DeviceIdType.MESH)
                pl.semaphore_signal(credit_l, inc=1, device_id=(right,),
                                    device_id_type=pl.DeviceIdType.MESH)
                pl.semaphore_wait(credit_r, 1)
                pl.semaphore_wait(credit_l, 1)
                mk_r(rslot, sslot, h + 1).start()
                mk_l(rslot, sslot, h + 1).start()

            tile_bufs[h, :, pl.ds(0, nh)] = jnp.dot(
                x_ref[:, :], comm_r[rslot],
                preferred_element_type=jnp.float32)
            tile_bufs[h, :, pl.ds(nh, nh)] = jnp.dot(
                x_ref[:, :], comm_l[rslot],
                preferred_element_type=jnp.float32)
            c_r = (my - h - 1) % N_DEV
            c_l = (my + h + 1) % N_DEV
            if h == 0:
                send_tile(tile_bufs.at[0, :, pl.ds(0, nh)], my, 0, nh, 0, c_r)
                send_tile(tile_bufs.at[0, :, pl.ds(nh, nh)], my, nh, nh, 1, c_l)
            elif h == 1:
                send_tile(tile_bufs.at[1], my, 0, 2 * nh, 2, c_r)
            else:
                send_tile(tile_bufs.at[2, :, pl.ds(0, nh)], my, 0, nh, 3, c_r)
                send_tile(tile_bufs.at[2, :, pl.ds(nh, nh)], my, nh, nh, 4, c_l)

        def wait_tile(src_rows, col, width, slot, src_dev):
            t = pltpu.make_async_remote_copy(
                src_ref=tile_bufs.at[0, :, pl.ds(0, width)],
                dst_ref=out_ref.at[pl.ds(src_rows * m_per, m_per),
                                   pl.ds(col, width)],
                send_sem=tile_send.at[slot],
                recv_sem=tile_recv.at[slot],
                device_id=(src_dev,),
                device_id_type=pl.DeviceIdType.MESH,
            )
            t.wait_send()
            t.wait_recv()

        wait_tile(right, 0, nh, 0, right)
        wait_tile(left, nh, nh, 1, left)
        wait_tile(diag, 0, 2 * nh, 2, diag)
        wait_tile(left, 0, nh, 3, left)
        wait_tile(right, nh, nh, 4, right)

    return pl.pallas_call(
        body,
        out_shape=jax.ShapeDtypeStruct((N_DEV * m_per, n_per), jnp.float32),
        in_specs=[
            pl.BlockSpec(memory_space=pltpu.VMEM),
            pl.BlockSpec(memory_space=pltpu.VMEM),
        ],
        out_specs=pl.BlockSpec(memory_space=pltpu.VMEM),
        scratch_shapes=[
            pltpu.VMEM((2, k, nh), jnp.bfloat16),
            pltpu.VMEM((2, k, nh), jnp.bfloat16),
            pltpu.VMEM((N_DEV - 1, m_per, n_per), jnp.float32),
            pltpu.SemaphoreType.DMA((N_DEV - 1,)),
            pltpu.SemaphoreType.DMA((N_DEV - 1,)),
            pltpu.SemaphoreType.DMA((N_DEV - 1,)),
            pltpu.SemaphoreType.DMA((N_DEV - 1,)),
            pltpu.SemaphoreType.REGULAR,
            pltpu.SemaphoreType.REGULAR,
            pltpu.SemaphoreType.DMA((5,)),
            pltpu.SemaphoreType.DMA((5,)),
        ],
        compiler_params=pltpu.CompilerParams(collective_id=0),
    )(xb, wb)


# device time: 112170 ns/iter; 5.1708x vs baseline; 1.1990x over previous
import jax
import jax.numpy as jnp
from jax import lax
from jax.experimental import pallas as pl
from jax.experimental.pallas import tpu as pltpu

N_DEV = 4


def kernel(x, w_mat):
    m_per, k = x.shape
    k2, n_per = w_mat.shape
    nh = n_per // 2

    xb = x.astype(jnp.bfloat16)
    wb = w_mat.astype(jnp.bfloat16)

    def body(x_ref, w_ref, out_ref, comm_r, comm_l, tile_bufs, tile_rx,
             send_r, recv_r, send_l, recv_l,
             credit_r, credit_l, tile_send, tile_recv):
        my = lax.axis_index("i")
        left = (my - 1) % N_DEV
        right = (my + 1) % N_DEV
        diag = (my + 2) % N_DEV

        barrier_sem = pltpu.get_barrier_semaphore()
        for nbr in [left, right, diag]:
            pl.semaphore_signal(
                barrier_sem, inc=1,
                device_id=(nbr,), device_id_type=pl.DeviceIdType.MESH,
            )
        pl.semaphore_wait(barrier_sem, 3)

        def mk(comm, sends, recvs, sslot, rslot, h, dst, src=None):
            return pltpu.make_async_remote_copy(
                src_ref=comm.at[sslot] if src is None else src,
                dst_ref=comm.at[rslot],
                send_sem=sends.at[h],
                recv_sem=recvs.at[h],
                device_id=(dst,),
                device_id_type=pl.DeviceIdType.MESH,
            )

        def mk_r(sslot, rslot, h, src=None):
            return mk(comm_r, send_r, recv_r, sslot, rslot, h, right, src)

        def mk_l(sslot, rslot, h, src=None):
            return mk(comm_l, send_l, recv_l, sslot, rslot, h, left, src)

        def send_tile(src, dst_rows, dst_col, dst_w, slot, dst_dev):
            t = pltpu.make_async_remote_copy(
                src_ref=src,
                dst_ref=tile_rx.at[dst_rows, :, pl.ds(dst_col, dst_w)],
                send_sem=tile_send.at[slot],
                recv_sem=tile_recv.at[slot],
                device_id=(dst_dev,),
                device_id_type=pl.DeviceIdType.MESH,
            )
            t.start()
            return t

        mk_r(0, 1, 0, src=w_ref.at[:, pl.ds(0, nh)]).start()
        mk_l(0, 1, 0, src=w_ref.at[:, pl.ds(nh, nh)]).start()

        out_ref[pl.ds(my * m_per, m_per), :] = jnp.dot(
            x_ref[:, :], w_ref[:, :], preferred_element_type=jnp.float32)

        for h in range(N_DEV - 1):
            sslot, rslot = h % 2, (h + 1) % 2
            r = mk_r(sslot, rslot, h)
            l = mk_l(sslot, rslot, h)
            r.wait_recv()
            l.wait_recv()
            r.wait_send()
            l.wait_send()
            if h < N_DEV - 2:
                pl.semaphore_signal(credit_r, inc=1, device_id=(left,),
                                    device_id_type=pl.DeviceIdType.MESH)
                pl.semaphore_signal(credit_l, inc=1, device_id=(right,),
                                    device_id_type=pl.DeviceIdType.MESH)
                pl.semaphore_wait(credit_r, 1)
                pl.semaphore_wait(credit_l, 1)
                mk_r(rslot, sslot, h + 1).start()
                mk_l(rslot, sslot, h + 1).start()

            tile_bufs[h, :, pl.ds(0, nh)] = jnp.dot(
                x_ref[:, :], comm_r[rslot],
                preferred_element_type=jnp.float32).astype(jnp.bfloat16)
            tile_bufs[h, :, pl.ds(nh, nh)] = jnp.dot(
                x_ref[:, :], comm_l[rslot],
                preferred_element_type=jnp.float32).astype(jnp.bfloat16)
            c_r = (my - h - 1) % N_DEV
            c_l = (my + h + 1) % N_DEV
            if h == 0:
                send_tile(tile_bufs.at[0, :, pl.ds(0, nh)], 0, 0, nh, 0, c_r)
                send_tile(tile_bufs.at[0, :, pl.ds(nh, nh)], 0, nh, nh, 1, c_l)
            elif h == 1:
                send_tile(tile_bufs.at[1], 1, 0, 2 * nh, 2, c_r)
            else:
                send_tile(tile_bufs.at[2, :, pl.ds(0, nh)], 2, 0, nh, 3, c_r)
                send_tile(tile_bufs.at[2, :, pl.ds(nh, nh)], 2, nh, nh, 4, c_l)

        def wait_tile(buf, col, width, slot, src_dev):
            t = pltpu.make_async_remote_copy(
                src_ref=tile_bufs.at[0, :, pl.ds(0, width)],
                dst_ref=tile_rx.at[buf, :, pl.ds(col, width)],
                send_sem=tile_send.at[slot],
                recv_sem=tile_recv.at[slot],
                device_id=(src_dev,),
                device_id_type=pl.DeviceIdType.MESH,
            )
            t.wait_send()
            t.wait_recv()

        wait_tile(0, 0, nh, 0, right)
        wait_tile(0, nh, nh, 1, left)
        wait_tile(1, 0, 2 * nh, 2, diag)
        wait_tile(2, 0, nh, 3, left)
        wait_tile(2, nh, nh, 4, right)

        out_ref[pl.ds(right * m_per, m_per), pl.ds(0, nh)] = (
            tile_rx[0, :, pl.ds(0, nh)].astype(jnp.float32))
        out_ref[pl.ds(left * m_per, m_per), pl.ds(nh, nh)] = (
            tile_rx[0, :, pl.ds(nh, nh)].astype(jnp.float32))
        out_ref[pl.ds(diag * m_per, m_per), :] = (
            tile_rx[1].astype(jnp.float32))
        out_ref[pl.ds(left * m_per, m_per), pl.ds(0, nh)] = (
            tile_rx[2, :, pl.ds(0, nh)].astype(jnp.float32))
        out_ref[pl.ds(right * m_per, m_per), pl.ds(nh, nh)] = (
            tile_rx[2, :, pl.ds(nh, nh)].astype(jnp.float32))

    return pl.pallas_call(
        body,
        out_shape=jax.ShapeDtypeStruct((N_DEV * m_per, n_per), jnp.float32),
        in_specs=[
            pl.BlockSpec(memory_space=pltpu.VMEM),
            pl.BlockSpec(memory_space=pltpu.VMEM),
        ],
        out_specs=pl.BlockSpec(memory_space=pltpu.VMEM),
        scratch_shapes=[
            pltpu.VMEM((2, k, nh), jnp.bfloat16),
            pltpu.VMEM((2, k, nh), jnp.bfloat16),
            pltpu.VMEM((N_DEV - 1, m_per, n_per), jnp.bfloat16),
            pltpu.VMEM((N_DEV - 1, m_per, n_per), jnp.bfloat16),
            pltpu.SemaphoreType.DMA((N_DEV - 1,)),
            pltpu.SemaphoreType.DMA((N_DEV - 1,)),
            pltpu.SemaphoreType.DMA((N_DEV - 1,)),
            pltpu.SemaphoreType.DMA((N_DEV - 1,)),
            pltpu.SemaphoreType.REGULAR,
            pltpu.SemaphoreType.REGULAR,
            pltpu.SemaphoreType.DMA((5,)),
            pltpu.SemaphoreType.DMA((5,)),
        ],
        compiler_params=pltpu.CompilerParams(collective_id=0),
    )(xb, wb)
